# Initial kernel scaffold; baseline (speedup 1.0000x reference)
import functools

import jax
import jax.numpy as jnp
from jax import lax
from jax.experimental import pallas as pl
from jax.experimental.pallas import tpu as pltpu
from jax.experimental.pallas import tpu_sc as plsc

_N = 10000
_E = 320000
_D = 128
_NC = 2
_NS = 16
_NW = _NC * _NS
_N_PAD = 10240
_E_PAD = 327680
_EPT = _E_PAD // _NW
_CHUNK = 128
_CPT = _EPT // _CHUNK
_RPT = _N_PAD // _NS

_BLK = 2048
_NBLK = _N_PAD // _BLK


def _sc_mesh():
    return plsc.VectorSubcoreMesh(core_axis_name="c", subcore_axis_name="s")


def _deg_partials(dst, ew):
    @functools.partial(
        pl.kernel,
        mesh=_sc_mesh(),
        out_type=jax.ShapeDtypeStruct((_NC, _N_PAD), jnp.float32),
        scratch_types=[
            pltpu.VMEM_SHARED((_N_PAD,), jnp.float32),
            pltpu.VMEM((_CHUNK,), jnp.int32),
            pltpu.VMEM((_CHUNK,), jnp.float32),
            pltpu.VMEM((_RPT,), jnp.float32),
        ],
    )
    def k(dst_hbm, ew_hbm, out_hbm, acc, idx_v, ew_v, zbuf):
        c = lax.axis_index("c")
        s = lax.axis_index("s")

        @pl.loop(0, _RPT // 16)
        def _(i):
            zbuf[pl.ds(i * 16, 16)] = jnp.zeros((16,), jnp.float32)

        pltpu.sync_copy(zbuf, acc.at[pl.ds(s * _RPT, _RPT)])
        plsc.subcore_barrier()

        base = (c * _NS + s) * _EPT

        @pl.loop(0, _CPT)
        def _(ch):
            b = base + ch * _CHUNK
            pltpu.sync_copy(dst_hbm.at[pl.ds(b, _CHUNK)], idx_v)
            pltpu.sync_copy(ew_hbm.at[pl.ds(b, _CHUNK)], ew_v)
            pltpu.sync_copy(ew_v, acc.at[idx_v], add=True)

        plsc.subcore_barrier()
        pltpu.sync_copy(acc.at[pl.ds(s * _RPT, _RPT)],
                        out_hbm.at[c, pl.ds(s * _RPT, _RPT)])

    return k(dst, ew)


def _scatter_partials(hp, src, dst, ew):
    @functools.partial(
        pl.kernel,
        mesh=_sc_mesh(),
        out_type=jax.ShapeDtypeStruct((_NC, _N_PAD, _D), jnp.float32),
        scratch_types=[
            pltpu.VMEM_SHARED((_N_PAD, _D), jnp.float32),
            pltpu.VMEM((_CHUNK,), jnp.int32),
            pltpu.VMEM((_CHUNK,), jnp.int32),
            pltpu.VMEM((_CHUNK,), jnp.float32),
            pltpu.VMEM((_CHUNK, _D), jnp.float32),
            pltpu.VMEM((_CHUNK, _D), jnp.float32),
            pltpu.SemaphoreType.DMA,
        ],
    )
    def k(hp_hbm, src_hbm, dst_hbm, ew_hbm, out_hbm,
          acc, src_v, dst_v, ew_v, rows_v, zblk, sem):
        c = lax.axis_index("c")
        s = lax.axis_index("s")

        @pl.loop(0, _CHUNK)
        def _(i):
            for f in range(_D // 16):
                zblk[i, pl.ds(f * 16, 16)] = jnp.zeros((16,), jnp.float32)

        @pl.loop(0, _RPT // _CHUNK)
        def _(kk):
            pltpu.sync_copy(zblk, acc.at[pl.ds(s * _RPT + kk * _CHUNK, _CHUNK)])

        plsc.subcore_barrier()

        base = (c * _NS + s) * _EPT

        @pl.loop(0, _CPT)
        def _(ch):
            b = base + ch * _CHUNK
            pltpu.sync_copy(src_hbm.at[pl.ds(b, _CHUNK)], src_v)
            pltpu.sync_copy(dst_hbm.at[pl.ds(b, _CHUNK)], dst_v)
            pltpu.sync_copy(ew_hbm.at[pl.ds(b, _CHUNK)], ew_v)
            pltpu.async_copy(hp_hbm.at[src_v], rows_v, sem).wait()

            @pl.loop(0, _CHUNK)
            def _(e):
                w16 = plsc.load_gather(ew_v, [jnp.full((16,), e, jnp.int32)])
                for f in range(_D // 16):
                    sl = pl.ds(f * 16, 16)
                    rows_v[e, sl] = rows_v[e, sl] * w16

            pltpu.sync_copy(rows_v, acc.at[dst_v], add=True)

        plsc.subcore_barrier()

        @pl.loop(0, _RPT // _CHUNK)
        def _(kk):
            r0 = s * _RPT + kk * _CHUNK
            pltpu.sync_copy(acc.at[pl.ds(r0, _CHUNK)],
                            out_hbm.at[c, pl.ds(r0, _CHUNK)])

    return k(hp, src, dst, ew)


def _dis_from_deg(deg_parts):
    deg_r = deg_parts.reshape(_NC, _N_PAD // 128, 128)

    def body(deg_ref, out_ref):
        out_ref[...] = lax.rsqrt(deg_ref[0] + deg_ref[1] + 1.0)

    return pl.pallas_call(
        body,
        out_shape=jax.ShapeDtypeStruct((_N_PAD // 128, 128), jnp.float32),
    )(deg_r)


def _mm_scale(x, W, dis):
    def body(x_ref, w_ref, dis_ref, o_ref):
        h = jnp.dot(x_ref[...], w_ref[...], preferred_element_type=jnp.float32)
        o_ref[...] = dis_ref[...] * h

    return pl.pallas_call(
        body,
        grid=(_NBLK,),
        in_specs=[
            pl.BlockSpec((_BLK, _D), lambda i: (i, 0)),
            pl.BlockSpec((_D, _D), lambda i: (0, 0)),
            pl.BlockSpec((_BLK, 1), lambda i: (i, 0)),
        ],
        out_specs=pl.BlockSpec((_BLK, _D), lambda i: (i, 0)),
        out_shape=jax.ShapeDtypeStruct((_N_PAD, _D), jnp.float32),
    )(x, W, dis)


def _layer2_mm(s_parts, hp, dis, W2, b1):
    def body(s_ref, hp_ref, dis_ref, w_ref, b_ref, o_ref):
        g = dis_ref[...] * (s_ref[0] + s_ref[1] + hp_ref[...]) + b_ref[...]
        g = jnp.maximum(g, 0.0)
        h2 = jnp.dot(g, w_ref[...], preferred_element_type=jnp.float32)
        o_ref[...] = dis_ref[...] * h2

    return pl.pallas_call(
        body,
        grid=(_NBLK,),
        in_specs=[
            pl.BlockSpec((_NC, _BLK, _D), lambda i: (0, i, 0)),
            pl.BlockSpec((_BLK, _D), lambda i: (i, 0)),
            pl.BlockSpec((_BLK, 1), lambda i: (i, 0)),
            pl.BlockSpec((_D, _D), lambda i: (0, 0)),
            pl.BlockSpec((1, _D), lambda i: (0, 0)),
        ],
        out_specs=pl.BlockSpec((_BLK, _D), lambda i: (i, 0)),
        out_shape=jax.ShapeDtypeStruct((_N_PAD, _D), jnp.float32),
    )(s_parts, hp, dis, W2, b1)


def _final_out(s_parts, hp, dis, b2):
    def body(s_ref, hp_ref, dis_ref, b_ref, o_ref):
        o_ref[...] = dis_ref[...] * (s_ref[0] + s_ref[1] + hp_ref[...]) + b_ref[...]

    return pl.pallas_call(
        body,
        grid=(_NBLK,),
        in_specs=[
            pl.BlockSpec((_NC, _BLK, _D), lambda i: (0, i, 0)),
            pl.BlockSpec((_BLK, _D), lambda i: (i, 0)),
            pl.BlockSpec((_BLK, 1), lambda i: (i, 0)),
            pl.BlockSpec((1, _D), lambda i: (0, 0)),
        ],
        out_specs=pl.BlockSpec((_BLK, _D), lambda i: (i, 0)),
        out_shape=jax.ShapeDtypeStruct((_N_PAD, _D), jnp.float32),
    )(s_parts, hp, dis, b2)


def kernel(x, edge_index, edge_weight, W1, b1, W2, b2):
    src = edge_index[0].astype(jnp.int32)
    dst = edge_index[1].astype(jnp.int32)
    ew = edge_weight.astype(jnp.float32)

    src = jnp.pad(src, (0, _E_PAD - _E))
    dst = jnp.pad(dst, (0, _E_PAD - _E))
    ew = jnp.pad(ew, (0, _E_PAD - _E))
    x_pad = jnp.pad(x, ((0, _N_PAD - _N), (0, 0)))

    deg_parts = _deg_partials(dst, ew)
    dis = _dis_from_deg(deg_parts).reshape(_N_PAD, 1)

    h1p = _mm_scale(x_pad, W1, dis)
    s1 = _scatter_partials(h1p, src, dst, ew)
    h2p = _layer2_mm(s1, h1p, dis, W2, b1.reshape(1, _D))
    s2 = _scatter_partials(h2p, src, dst, ew)
    out = _final_out(s2, h2p, dis, b2.reshape(1, _D))

    return out[:_N]


# speedup vs baseline: 6.8329x; 6.8329x over previous
import dataclasses
import functools

import jax
import jax.numpy as jnp
from jax import lax
from jax.experimental import pallas as pl
from jax.experimental.pallas import tpu as pltpu
from jax.experimental.pallas import tpu_sc as plsc

_N = 10000
_E = 320000
_D = 128
_NC = 2
_NS = 16
_NW = _NC * _NS
_N_PAD = 10240
_E_PAD = 327680
_EPT = _E_PAD // _NW
_CHUNK = 128
_CPT = _EPT // _CHUNK
_RPT = _N_PAD // _NS

_BLK = 2048
_NBLK = _N_PAD // _BLK


def _sc_mesh():
    return plsc.VectorSubcoreMesh(core_axis_name="c", subcore_axis_name="s")


def _sc_compiler_params():
    cp = pltpu.CompilerParams()
    if "needs_layout_passes" in pltpu.CompilerParams.__dataclass_fields__:
        cp = dataclasses.replace(cp, needs_layout_passes=False)
    return cp


def _deg_partials(dst, ew):
    @functools.partial(
        pl.kernel,
        mesh=_sc_mesh(),
        out_type=jax.ShapeDtypeStruct((_NC, _N_PAD), jnp.float32),
        scratch_types=[
            pltpu.VMEM_SHARED((_N_PAD,), jnp.float32),
            pltpu.VMEM((_CHUNK,), jnp.int32),
            pltpu.VMEM((_CHUNK,), jnp.float32),
            pltpu.VMEM((_RPT,), jnp.float32),
        ],
    )
    def k(dst_hbm, ew_hbm, out_hbm, acc, idx_v, ew_v, zbuf):
        c = lax.axis_index("c")
        s = lax.axis_index("s")

        @pl.loop(0, _RPT // 16)
        def _(i):
            zbuf[pl.ds(i * 16, 16)] = jnp.zeros((16,), jnp.float32)

        pltpu.sync_copy(zbuf, acc.at[pl.ds(s * _RPT, _RPT)])
        plsc.subcore_barrier()

        base = (c * _NS + s) * _EPT

        @pl.loop(0, _CPT)
        def _(ch):
            b = base + ch * _CHUNK
            pltpu.sync_copy(dst_hbm.at[pl.ds(b, _CHUNK)], idx_v)
            pltpu.sync_copy(ew_hbm.at[pl.ds(b, _CHUNK)], ew_v)
            pltpu.sync_copy(ew_v, acc.at[idx_v], add=True)

        plsc.subcore_barrier()
        pltpu.sync_copy(acc.at[pl.ds(s * _RPT, _RPT)],
                        out_hbm.at[c, pl.ds(s * _RPT, _RPT)])

    return k(dst, ew)


def _scatter_partials(hp, src, dst, ew):
    @functools.partial(
        pl.kernel,
        mesh=_sc_mesh(),
        out_type=jax.ShapeDtypeStruct((_NC, _N_PAD, _D), jnp.float32),
        compiler_params=_sc_compiler_params(),
        scratch_types=[
            pltpu.VMEM_SHARED((_N_PAD, _D), jnp.float32),
            pltpu.VMEM((_CHUNK,), jnp.int32),
            pltpu.VMEM((_CHUNK,), jnp.int32),
            pltpu.VMEM((_CHUNK,), jnp.float32),
            pltpu.VMEM((_CHUNK, _D), jnp.float32),
            pltpu.VMEM((_CHUNK, _D), jnp.float32),
            pltpu.SemaphoreType.DMA,
        ],
    )
    def k(hp_hbm, src_hbm, dst_hbm, ew_hbm, out_hbm,
          acc, src_v, dst_v, ew_v, rows_v, zblk, sem):
        c = lax.axis_index("c")
        s = lax.axis_index("s")

        @pl.loop(0, _CHUNK)
        def _(i):
            for f in range(_D // 16):
                zblk[i, pl.ds(f * 16, 16)] = jnp.zeros((16,), jnp.float32)

        @pl.loop(0, _RPT // _CHUNK)
        def _(kk):
            pltpu.sync_copy(zblk, acc.at[pl.ds(s * _RPT + kk * _CHUNK, _CHUNK)])

        plsc.subcore_barrier()

        base = (c * _NS + s) * _EPT

        @pl.loop(0, _CPT)
        def _(ch):
            b = base + ch * _CHUNK
            pltpu.sync_copy(src_hbm.at[pl.ds(b, _CHUNK)], src_v)
            pltpu.sync_copy(dst_hbm.at[pl.ds(b, _CHUNK)], dst_v)
            pltpu.sync_copy(ew_hbm.at[pl.ds(b, _CHUNK)], ew_v)
            pltpu.async_copy(hp_hbm.at[src_v], rows_v, sem).wait()

            @pl.loop(0, _CHUNK)
            def _(e):
                w16 = plsc.load_gather(ew_v, [jnp.full((16,), e, jnp.int32)])
                for f in range(_D // 16):
                    sl = pl.ds(f * 16, 16)
                    rows_v[e, sl] = rows_v[e, sl] * w16

            pltpu.sync_copy(rows_v, acc.at[dst_v], add=True)

        plsc.subcore_barrier()

        @pl.loop(0, _RPT // _CHUNK)
        def _(kk):
            r0 = s * _RPT + kk * _CHUNK
            pltpu.sync_copy(acc.at[pl.ds(r0, _CHUNK)],
                            out_hbm.at[c, pl.ds(r0, _CHUNK)])

    return k(hp, src, dst, ew)


def _dis_from_deg(deg_parts):
    deg_r = deg_parts.reshape(_NC, _N_PAD // 128, 128)

    def body(deg_ref, out_ref):
        out_ref[...] = lax.rsqrt(deg_ref[0] + deg_ref[1] + 1.0)

    return pl.pallas_call(
        body,
        out_shape=jax.ShapeDtypeStruct((_N_PAD // 128, 128), jnp.float32),
    )(deg_r)


def _mm_scale(x, W, dis):
    def body(x_ref, w_ref, dis_ref, o_ref):
        h = jnp.dot(x_ref[...], w_ref[...], preferred_element_type=jnp.float32)
        o_ref[...] = dis_ref[...] * h

    return pl.pallas_call(
        body,
        grid=(_NBLK,),
        in_specs=[
            pl.BlockSpec((_BLK, _D), lambda i: (i, 0)),
            pl.BlockSpec((_D, _D), lambda i: (0, 0)),
            pl.BlockSpec((_BLK, 1), lambda i: (i, 0)),
        ],
        out_specs=pl.BlockSpec((_BLK, _D), lambda i: (i, 0)),
        out_shape=jax.ShapeDtypeStruct((_N_PAD, _D), jnp.float32),
    )(x, W, dis)


def _layer2_mm(s_parts, hp, dis, W2, b1):
    def body(s_ref, hp_ref, dis_ref, w_ref, b_ref, o_ref):
        g = dis_ref[...] * (s_ref[0] + s_ref[1] + hp_ref[...]) + b_ref[...]
        g = jnp.maximum(g, 0.0)
        h2 = jnp.dot(g, w_ref[...], preferred_element_type=jnp.float32)
        o_ref[...] = dis_ref[...] * h2

    return pl.pallas_call(
        body,
        grid=(_NBLK,),
        in_specs=[
            pl.BlockSpec((_NC, _BLK, _D), lambda i: (0, i, 0)),
            pl.BlockSpec((_BLK, _D), lambda i: (i, 0)),
            pl.BlockSpec((_BLK, 1), lambda i: (i, 0)),
            pl.BlockSpec((_D, _D), lambda i: (0, 0)),
            pl.BlockSpec((1, _D), lambda i: (0, 0)),
        ],
        out_specs=pl.BlockSpec((_BLK, _D), lambda i: (i, 0)),
        out_shape=jax.ShapeDtypeStruct((_N_PAD, _D), jnp.float32),
    )(s_parts, hp, dis, W2, b1)


def _final_out(s_parts, hp, dis, b2):
    def body(s_ref, hp_ref, dis_ref, b_ref, o_ref):
        o_ref[...] = dis_ref[...] * (s_ref[0] + s_ref[1] + hp_ref[...]) + b_ref[...]

    return pl.pallas_call(
        body,
        grid=(_NBLK,),
        in_specs=[
            pl.BlockSpec((_NC, _BLK, _D), lambda i: (0, i, 0)),
            pl.BlockSpec((_BLK, _D), lambda i: (i, 0)),
            pl.BlockSpec((_BLK, 1), lambda i: (i, 0)),
            pl.BlockSpec((1, _D), lambda i: (0, 0)),
        ],
        out_specs=pl.BlockSpec((_BLK, _D), lambda i: (i, 0)),
        out_shape=jax.ShapeDtypeStruct((_N_PAD, _D), jnp.float32),
    )(s_parts, hp, dis, b2)


def kernel(x, edge_index, edge_weight, W1, b1, W2, b2):
    src = edge_index[0].astype(jnp.int32)
    dst = edge_index[1].astype(jnp.int32)
    ew = edge_weight.astype(jnp.float32)

    src = jnp.pad(src, (0, _E_PAD - _E))
    dst = jnp.pad(dst, (0, _E_PAD - _E))
    ew = jnp.pad(ew, (0, _E_PAD - _E))
    x_pad = jnp.pad(x, ((0, _N_PAD - _N), (0, 0)))

    deg_parts = _deg_partials(dst, ew)
    dis = _dis_from_deg(deg_parts).reshape(_N_PAD, 1)

    h1p = _mm_scale(x_pad, W1, dis)
    s1 = _scatter_partials(h1p, src, dst, ew)
    h2p = _layer2_mm(s1, h1p, dis, W2, b1.reshape(1, _D))
    s2 = _scatter_partials(h2p, src, dst, ew)
    out = _final_out(s2, h2p, dis, b2.reshape(1, _D))

    return out[:_N]


# speedup vs baseline: 10.5106x; 1.5382x over previous
import dataclasses
import functools

import jax
import jax.numpy as jnp
from jax import lax
from jax.experimental import pallas as pl
from jax.experimental.pallas import tpu as pltpu
from jax.experimental.pallas import tpu_sc as plsc

_N = 10000
_E = 320000
_D = 128
_NC = 2
_NS = 16
_NW = _NC * _NS
_N_PAD = 10240
_E_PAD = 327680
_EPT = _E_PAD // _NW
_CHUNK = 128
_CPT = _EPT // _CHUNK
_RPT = _N_PAD // _NS

_BLK = 2048
_NBLK = _N_PAD // _BLK


def _sc_mesh():
    return plsc.VectorSubcoreMesh(core_axis_name="c", subcore_axis_name="s")


def _sc_compiler_params():
    cp = pltpu.CompilerParams()
    if "needs_layout_passes" in pltpu.CompilerParams.__dataclass_fields__:
        cp = dataclasses.replace(cp, needs_layout_passes=False)
    return cp


def _deg_partials(dst_r, ew_r):
    grp = 16

    @functools.partial(
        pl.kernel,
        mesh=_sc_mesh(),
        out_type=jax.ShapeDtypeStruct((_NC, _N_PAD), jnp.float32),
        scratch_types=[
            pltpu.VMEM_SHARED((_N_PAD,), jnp.float32),
            pltpu.VMEM((_CPT, _CHUNK), jnp.int32),
            pltpu.VMEM((_CPT, _CHUNK), jnp.float32),
            pltpu.VMEM((_RPT,), jnp.float32),
            pltpu.SemaphoreType.DMA,
        ],
    )
    def k(dst_hbm, ew_hbm, out_hbm, acc, idx_all, ew_all, zbuf, sem):
        c = lax.axis_index("c")
        s = lax.axis_index("s")
        wid = c * _NS + s

        pltpu.sync_copy(dst_hbm.at[wid], idx_all)
        pltpu.sync_copy(ew_hbm.at[wid], ew_all)

        @pl.loop(0, _RPT // 16)
        def _(i):
            zbuf[pl.ds(i * 16, 16)] = jnp.zeros((16,), jnp.float32)

        pltpu.sync_copy(zbuf, acc.at[pl.ds(s * _RPT, _RPT)])
        plsc.subcore_barrier()

        @pl.loop(0, _CPT // grp)
        def _(gi):
            for j in range(grp):
                pltpu.async_copy(ew_all.at[gi * grp + j],
                                 acc.at[idx_all.at[gi * grp + j]], sem,
                                 add=True)
            for j in range(grp):
                pltpu.make_async_copy(ew_all.at[gi * grp + j],
                                      acc.at[idx_all.at[gi * grp + j]],
                                      sem).wait()

        plsc.subcore_barrier()
        pltpu.sync_copy(acc.at[pl.ds(s * _RPT, _RPT)],
                        out_hbm.at[c, pl.ds(s * _RPT, _RPT)])

    return k(dst_r, ew_r)


def _scatter_partials(hp, src_r, dst_r, ew_r):
    @functools.partial(
        pl.kernel,
        mesh=_sc_mesh(),
        out_type=jax.ShapeDtypeStruct((_NC, _N_PAD, _D), jnp.float32),
        compiler_params=_sc_compiler_params(),
        scratch_types=[
            pltpu.VMEM_SHARED((_N_PAD, _D), jnp.float32),
            pltpu.VMEM((_CHUNK,), jnp.int32),
            pltpu.VMEM((_CHUNK,), jnp.int32),
            pltpu.VMEM((_CHUNK,), jnp.int32),
            pltpu.VMEM((_CHUNK,), jnp.int32),
            pltpu.VMEM((_CHUNK,), jnp.float32),
            pltpu.VMEM((_CHUNK,), jnp.float32),
            pltpu.VMEM((_CHUNK, _D), jnp.float32),
            pltpu.VMEM((_CHUNK, _D), jnp.float32),
            pltpu.SemaphoreType.DMA,
            pltpu.SemaphoreType.DMA,
        ],
    )
    def k(hp_hbm, src_hbm, dst_hbm, ew_hbm, out_hbm,
          acc, sv0, sv1, dv0, dv1, wv0, wv1, b0, b1, gs0, gs1):
        c = lax.axis_index("c")
        s = lax.axis_index("s")
        wid = c * _NS + s
        bufs = (b0, b1)
        srcv = (sv0, sv1)
        dstv = (dv0, dv1)
        eww = (wv0, wv1)
        sems = (gs0, gs1)

        @pl.loop(0, _CHUNK)
        def _(i):
            for f in range(_D // 16):
                b0[i, pl.ds(f * 16, 16)] = jnp.zeros((16,), jnp.float32)

        @pl.loop(0, _RPT // _CHUNK)
        def _(kk):
            pltpu.sync_copy(b0, acc.at[pl.ds(s * _RPT + kk * _CHUNK, _CHUNK)])

        plsc.subcore_barrier()

        def fetch(g, par):
            pltpu.async_copy(src_hbm.at[wid, g], srcv[par], sems[par])
            pltpu.async_copy(dst_hbm.at[wid, g], dstv[par], sems[par])
            pltpu.async_copy(ew_hbm.at[wid, g], eww[par], sems[par])

        def wait_fetch(g, par):
            pltpu.make_async_copy(src_hbm.at[wid, g], srcv[par],
                                  sems[par]).wait()
            pltpu.make_async_copy(dst_hbm.at[wid, g], dstv[par],
                                  sems[par]).wait()
            pltpu.make_async_copy(ew_hbm.at[wid, g], eww[par],
                                  sems[par]).wait()

        def scale(buf, par):
            @pl.loop(0, _CHUNK)
            def _(e):
                w16 = plsc.load_gather(eww[par],
                                       [jnp.full((16,), e, jnp.int32)])
                for f in range(_D // 16):
                    sl = pl.ds(f * 16, 16)
                    buf[e, sl] = buf[e, sl] * w16

        fetch(0, 0)
        wait_fetch(0, 0)
        pltpu.async_copy(hp_hbm.at[srcv[0]], b0, sems[0])
        fetch(1, 1)

        @pl.loop(0, _CPT // 2)
        def _(i):
            for par in range(2):
                g = 2 * i + par
                nxt = jnp.minimum(g + 1, _CPT - 1)
                nx2 = jnp.minimum(g + 2, _CPT - 1)
                wait_fetch(nxt, 1 - par)
                pltpu.async_copy(hp_hbm.at[srcv[1 - par]],
                                 bufs[1 - par], sems[1 - par])
                pltpu.make_async_copy(hp_hbm.at[srcv[par]],
                                      bufs[par], sems[par]).wait()
                scale(bufs[par], par)
                pltpu.sync_copy(bufs[par], acc.at[dstv[par]], add=True)
                fetch(nx2, par)

        wait_fetch(_CPT - 1, 1)
        pltpu.make_async_copy(hp_hbm.at[srcv[0]], b0, sems[0]).wait()

        plsc.subcore_barrier()

        @pl.loop(0, _RPT // _CHUNK)
        def _(kk):
            r0 = s * _RPT + kk * _CHUNK
            pltpu.sync_copy(acc.at[pl.ds(r0, _CHUNK)],
                            out_hbm.at[c, pl.ds(r0, _CHUNK)])

    return k(hp, src_r, dst_r, ew_r)


def _dis_from_deg(deg_parts):
    deg_r = deg_parts.reshape(_NC, _N_PAD // 128, 128)

    def body(deg_ref, out_ref):
        out_ref[...] = lax.rsqrt(deg_ref[0] + deg_ref[1] + 1.0)

    return pl.pallas_call(
        body,
        out_shape=jax.ShapeDtypeStruct((_N_PAD // 128, 128), jnp.float32),
    )(deg_r)


def _mm_scale(x, W, dis):
    def body(x_ref, w_ref, dis_ref, o_ref):
        h = jnp.dot(x_ref[...], w_ref[...], preferred_element_type=jnp.float32)
        o_ref[...] = dis_ref[...] * h

    return pl.pallas_call(
        body,
        grid=(_NBLK,),
        in_specs=[
            pl.BlockSpec((_BLK, _D), lambda i: (i, 0)),
            pl.BlockSpec((_D, _D), lambda i: (0, 0)),
            pl.BlockSpec((_BLK, 1), lambda i: (i, 0)),
        ],
        out_specs=pl.BlockSpec((_BLK, _D), lambda i: (i, 0)),
        out_shape=jax.ShapeDtypeStruct((_N_PAD, _D), jnp.float32),
    )(x, W, dis)


def _layer2_mm(s_parts, hp, dis, W2, b1):
    def body(s_ref, hp_ref, dis_ref, w_ref, b_ref, o_ref):
        g = dis_ref[...] * (s_ref[0] + s_ref[1] + hp_ref[...]) + b_ref[...]
        g = jnp.maximum(g, 0.0)
        h2 = jnp.dot(g, w_ref[...], preferred_element_type=jnp.float32)
        o_ref[...] = dis_ref[...] * h2

    return pl.pallas_call(
        body,
        grid=(_NBLK,),
        in_specs=[
            pl.BlockSpec((_NC, _BLK, _D), lambda i: (0, i, 0)),
            pl.BlockSpec((_BLK, _D), lambda i: (i, 0)),
            pl.BlockSpec((_BLK, 1), lambda i: (i, 0)),
            pl.BlockSpec((_D, _D), lambda i: (0, 0)),
            pl.BlockSpec((1, _D), lambda i: (0, 0)),
        ],
        out_specs=pl.BlockSpec((_BLK, _D), lambda i: (i, 0)),
        out_shape=jax.ShapeDtypeStruct((_N_PAD, _D), jnp.float32),
    )(s_parts, hp, dis, W2, b1)


def _final_out(s_parts, hp, dis, b2):
    def body(s_ref, hp_ref, dis_ref, b_ref, o_ref):
        o_ref[...] = dis_ref[...] * (s_ref[0] + s_ref[1] + hp_ref[...]) + b_ref[...]

    return pl.pallas_call(
        body,
        grid=(_NBLK,),
        in_specs=[
            pl.BlockSpec((_NC, _BLK, _D), lambda i: (0, i, 0)),
            pl.BlockSpec((_BLK, _D), lambda i: (i, 0)),
            pl.BlockSpec((_BLK, 1), lambda i: (i, 0)),
            pl.BlockSpec((1, _D), lambda i: (0, 0)),
        ],
        out_specs=pl.BlockSpec((_BLK, _D), lambda i: (i, 0)),
        out_shape=jax.ShapeDtypeStruct((_N_PAD, _D), jnp.float32),
    )(s_parts, hp, dis, b2)


def kernel(x, edge_index, edge_weight, W1, b1, W2, b2):
    src = edge_index[0].astype(jnp.int32)
    dst = edge_index[1].astype(jnp.int32)
    ew = edge_weight.astype(jnp.float32)

    src = jnp.pad(src, (0, _E_PAD - _E)).reshape(_NW, _CPT, _CHUNK)
    dst = jnp.pad(dst, (0, _E_PAD - _E)).reshape(_NW, _CPT, _CHUNK)
    ew = jnp.pad(ew, (0, _E_PAD - _E)).reshape(_NW, _CPT, _CHUNK)
    x_pad = jnp.pad(x, ((0, _N_PAD - _N), (0, 0)))

    deg_parts = _deg_partials(dst, ew)
    dis = _dis_from_deg(deg_parts).reshape(_N_PAD, 1)

    h1p = _mm_scale(x_pad, W1, dis)
    s1 = _scatter_partials(h1p, src, dst, ew)
    h2p = _layer2_mm(s1, h1p, dis, W2, b1.reshape(1, _D))
    s2 = _scatter_partials(h2p, src, dst, ew)
    out = _final_out(s2, h2p, dis, b2.reshape(1, _D))

    return out[:_N]
